# baseline (speedup 1.0000x reference)
import jax
import jax.numpy as jnp
from jax.experimental import pallas as pl

NUM_FEATURES = 2048
BR = 512


LANES = 128
NT = NUM_FEATURES // LANES


def _permute_block(x_ref, idx_ref, z_ref):
    x = x_ref[...]
    br = x.shape[0]
    for t in range(NT):
        it = idx_ref[0:1, t * LANES:(t + 1) * LANES]
        lane = jnp.broadcast_to(it % LANES, (br, LANES))
        src = it // LANES
        acc = None
        for s in range(NT):
            g = jnp.take_along_axis(
                x[:, s * LANES:(s + 1) * LANES], lane, axis=1)
            m = src == s
            acc = jnp.where(m, g, 0.0) if acc is None else jnp.where(m, g, acc)
        z_ref[:, t * LANES:(t + 1) * LANES] = acc


def kernel(x, idx):
    n, f = x.shape
    idx2d = idx.reshape(1, f)
    grid = (n // BR,)
    z = pl.pallas_call(
        _permute_block,
        grid=grid,
        in_specs=[
            pl.BlockSpec((BR, f), lambda i: (i, 0)),
            pl.BlockSpec((1, f), lambda i: (0, 0)),
        ],
        out_specs=pl.BlockSpec((BR, f), lambda i: (i, 0)),
        out_shape=jax.ShapeDtypeStruct((n, f), x.dtype),
    )(x, idx2d)
    logdet = jnp.zeros((n,), dtype=x.dtype)
    return (z, logdet)
